# Initial kernel scaffold; baseline (speedup 1.0000x reference)
import jax
import jax.numpy as jnp
from jax.experimental import pallas as pl


def kernel(x, edge_index, edge_attr, batch, W_emb, b_emb, W1, b1, W2, b2, W_ih, W_hh, b_ih, b_hh, bn_gamma, bn_beta):
    raise NotImplementedError("write your pallas kernel here")


# speedup vs baseline: 1.0057x; 1.0057x over previous
import functools

import jax
import jax.numpy as jnp
from jax import lax
from jax.experimental import pallas as pl
from jax.experimental.pallas import tpu as pltpu
from jax.experimental.pallas import tpu_sc as plsc

N = 10000
E = 320000
H = 256
ED = 4
G = 128
L = 5

NW = 32
NPC = N // 2

EPW = E // NW
CG = 80
EPT = E // 16
CS = 80

PPW = 312
CP = 104

_MESH = plsc.VectorSubcoreMesh(core_axis_name="c", subcore_axis_name="s")


def _emb_body(x_ref, we_ref, be_ref, w1a_ref, h_ref, p_ref):
    h = jnp.dot(x_ref[...], we_ref[...], preferred_element_type=jnp.float32)
    h = h + be_ref[...]
    h_ref[...] = h
    p_ref[...] = jnp.dot(h, w1a_ref[...], preferred_element_type=jnp.float32)


def _embed(x, W_emb, b_emb, w1a0):
    return pl.pallas_call(
        _emb_body,
        out_shape=(jax.ShapeDtypeStruct((N, H), jnp.float32),
                   jax.ShapeDtypeStruct((N, H), jnp.float32)),
    )(x, W_emb, b_emb.reshape(1, H), w1a0)


BE = 512


def _edge_body(pj_ref, ea_ref, w1b_ref, b1_ref, w2_ref, b2_ref, m_ref):
    t = pj_ref[...] + jnp.dot(ea_ref[...], w1b_ref[...],
                              preferred_element_type=jnp.float32) + b1_ref[...]
    t = jnp.maximum(t, 0.0)
    m_ref[...] = jnp.dot(t, w2_ref[...],
                         preferred_element_type=jnp.float32) + b2_ref[...]


def _edge_mlp(pj, ea, w1b, b1, w2, b2):
    return pl.pallas_call(
        _edge_body,
        grid=(E // BE,),
        in_specs=[pl.BlockSpec((BE, H), lambda i: (i, 0)),
                  pl.BlockSpec((BE, ED), lambda i: (i, 0)),
                  pl.BlockSpec((ED, H), lambda i: (0, 0)),
                  pl.BlockSpec((1, H), lambda i: (0, 0)),
                  pl.BlockSpec((H, H), lambda i: (0, 0)),
                  pl.BlockSpec((1, H), lambda i: (0, 0))],
        out_specs=pl.BlockSpec((BE, H), lambda i: (i, 0)),
        out_shape=jax.ShapeDtypeStruct((E, H), jnp.float32),
    )(pj, ea, w1b, b1.reshape(1, H), w2, b2.reshape(1, H))


BN = 1000
NB = N // BN


def _gru1_body(aggr_ref, h_ref, wih_ref, whh_ref, bih_ref, bhh_ref,
               hn_ref, ps_ref, pq_ref):
    aggr = aggr_ref[...]
    h = h_ref[...]
    gi = jnp.dot(aggr, wih_ref[...],
                 preferred_element_type=jnp.float32) + bih_ref[...]
    gh = jnp.dot(h, whh_ref[...],
                 preferred_element_type=jnp.float32) + bhh_ref[...]
    r = jax.nn.sigmoid(gi[:, :H] + gh[:, :H])
    z = jax.nn.sigmoid(gi[:, H:2 * H] + gh[:, H:2 * H])
    n = jnp.tanh(gi[:, 2 * H:] + r * gh[:, 2 * H:])
    hn = (1.0 - z) * n + z * h
    hn_ref[...] = hn
    ps_ref[...] = jnp.sum(hn, axis=0, keepdims=True).reshape(1, 1, H)
    pq_ref[...] = jnp.sum(hn * hn, axis=0, keepdims=True).reshape(1, 1, H)


def _gru2_body(hn_ref, ps_ref, pq_ref, gam_ref, bet_ref, w1a_ref,
               hout_ref, pout_ref):
    mu = jnp.sum(ps_ref[...], axis=0) / N
    msq = jnp.sum(pq_ref[...], axis=0) / N
    var = msq - mu * mu
    inv = lax.rsqrt(var + 1e-5)
    y = (hn_ref[...] - mu) * inv * gam_ref[...] + bet_ref[...]
    ho = jnp.maximum(y, 0.0)
    hout_ref[...] = ho
    pout_ref[...] = jnp.dot(ho, w1a_ref[...],
                            preferred_element_type=jnp.float32)


def _gru(aggr, h, wihT, whhT, bih, bhh, gam, bet, w1a_next):
    hn, ps, pq = pl.pallas_call(
        _gru1_body,
        grid=(NB,),
        in_specs=[pl.BlockSpec((BN, H), lambda i: (i, 0)),
                  pl.BlockSpec((BN, H), lambda i: (i, 0)),
                  pl.BlockSpec((H, 3 * H), lambda i: (0, 0)),
                  pl.BlockSpec((H, 3 * H), lambda i: (0, 0)),
                  pl.BlockSpec((1, 3 * H), lambda i: (0, 0)),
                  pl.BlockSpec((1, 3 * H), lambda i: (0, 0))],
        out_specs=(pl.BlockSpec((BN, H), lambda i: (i, 0)),
                   pl.BlockSpec((1, 1, H), lambda i: (i, 0, 0)),
                   pl.BlockSpec((1, 1, H), lambda i: (i, 0, 0))),
        out_shape=(jax.ShapeDtypeStruct((N, H), jnp.float32),
                   jax.ShapeDtypeStruct((NB, 1, H), jnp.float32),
                   jax.ShapeDtypeStruct((NB, 1, H), jnp.float32)),
    )(aggr, h, wihT, whhT, bih.reshape(1, 3 * H), bhh.reshape(1, 3 * H))

    return pl.pallas_call(
        _gru2_body,
        grid=(NB,),
        in_specs=[pl.BlockSpec((BN, H), lambda i: (i, 0)),
                  pl.BlockSpec((NB, 1, H), lambda i: (0, 0, 0)),
                  pl.BlockSpec((NB, 1, H), lambda i: (0, 0, 0)),
                  pl.BlockSpec((1, H), lambda i: (0, 0)),
                  pl.BlockSpec((1, H), lambda i: (0, 0)),
                  pl.BlockSpec((H, H), lambda i: (0, 0))],
        out_specs=(pl.BlockSpec((BN, H), lambda i: (i, 0)),
                   pl.BlockSpec((BN, H), lambda i: (i, 0))),
        out_shape=(jax.ShapeDtypeStruct((N, H), jnp.float32),
                   jax.ShapeDtypeStruct((N, H), jnp.float32)),
    )(hn, ps, pq, gam.reshape(1, H), bet.reshape(1, H), w1a_next)


def _combine_body(psum_ref, pmax_ref, bat_ref, out_ref):
    sums = jnp.sum(psum_ref[...], axis=0)
    maxr = jnp.max(pmax_ref[...], axis=0)
    oh = (bat_ref[...] == lax.broadcasted_iota(jnp.int32, (G, N), 0))
    cnt = jnp.sum(oh.astype(jnp.float32), axis=1, keepdims=True)
    mean = sums / jnp.maximum(cnt, 1.0)
    out_ref[:, :H] = mean
    out_ref[:, H:] = maxr


def _combine(psum, pmax, batch1n):
    return pl.pallas_call(
        _combine_body,
        out_shape=jax.ShapeDtypeStruct((G, 2 * H), jnp.float32),
    )(psum, pmax, batch1n)


@functools.partial(
    pl.kernel, mesh=_MESH,
    out_type=jax.ShapeDtypeStruct((E, H), jnp.float32),
    scratch_types=[pltpu.VMEM((CG,), jnp.int32),
                   pltpu.VMEM((CG, H), jnp.float32),
                   pltpu.SemaphoreType.DMA],
)
def _gather_k(p_hbm, src_hbm, out_hbm, idx_v, rows_v, sem):
    wid = lax.axis_index("s") * 2 + lax.axis_index("c")
    base = wid * EPW

    def body(i, carry):
        off = base + i * CG
        pltpu.sync_copy(src_hbm.at[pl.ds(off, CG)], idx_v)
        pltpu.async_copy(p_hbm.at[idx_v], rows_v, sem).wait()
        pltpu.sync_copy(rows_v, out_hbm.at[pl.ds(off, CG)])
        return carry

    lax.fori_loop(0, EPW // CG, body, 0)


OWN = 312
OWN_LAST = N - 31 * OWN
ACC_ROWS = 336
TRASH = 330
CAP = 13056
FB = 64
CD = 1024


def _prep_lists(dst):
    eid = jnp.arange(E, dtype=jnp.int32)
    owner = jnp.minimum(dst // OWN, NW - 1)
    lid = dst - owner * OWN
    pk = eid * 512 + lid
    order = jnp.argsort(owner, stable=True)
    owner_s = owner[order]
    pk_s = pk[order]
    counts = jax.ops.segment_sum(jnp.ones((E,), jnp.int32), owner,
                                 num_segments=NW)
    starts = jnp.concatenate(
        [jnp.zeros((1,), jnp.int32), jnp.cumsum(counts)[:-1]])
    within = eid - starts[owner_s]
    lists = jnp.full((NW, CAP + FB + 16), TRASH, jnp.int32)
    lists = lists.at[owner_s, within].set(pk_s)
    ecnts = jnp.broadcast_to(counts[:, None], (NW, 16)).astype(jnp.int32)
    return lists, ecnts


@functools.partial(
    pl.kernel, mesh=_MESH,
    out_type=jax.ShapeDtypeStruct((N, H), jnp.float32),
    scratch_types=[pltpu.VMEM((CAP + FB + 16,), jnp.int32),
                   pltpu.VMEM((16,), jnp.int32),
                   pltpu.VMEM((FB,), jnp.int32),
                   pltpu.VMEM((FB + 16,), jnp.int32),
                   pltpu.VMEM((FB, H), jnp.float32),
                   pltpu.VMEM((ACC_ROWS, H), jnp.float32),
                   pltpu.SemaphoreType.DMA],
)
def _scatter_k(m_hbm, list_hbm, cnt_hbm, zero_hbm, out_hbm,
               buf_v, cnt_v, eid_v, lid_v, rows_v, acc_v, sem):
    wid = lax.axis_index("s") * 2 + lax.axis_index("c")
    base = wid * OWN

    pltpu.sync_copy(list_hbm.at[wid], buf_v)
    pltpu.sync_copy(cnt_hbm.at[wid], cnt_v)
    cnt = cnt_v[pl.ds(0, 16)][0]

    for k in range(4):
        pltpu.sync_copy(zero_hbm.at[pl.ds(0, CS)],
                        acc_v.at[pl.ds(k * CS, CS)])
    pltpu.sync_copy(zero_hbm.at[pl.ds(0, ACC_ROWS - 4 * CS)],
                    acc_v.at[pl.ds(4 * CS, ACC_ROWS - 4 * CS)])

    nb = (cnt + FB - 1) // FB

    def block(b, carry):
        def unpack(u, c2):
            pk = buf_v[pl.ds(b * FB + u * 16, 16)]
            eid_v[pl.ds(u * 16, 16)] = lax.shift_right_logical(pk, 9)
            lid_v[pl.ds(u * 16, 16)] = pk & 511
            return c2

        lax.fori_loop(0, FB // 16, unpack, 0)
        pltpu.async_copy(m_hbm.at[eid_v], rows_v, sem).wait()

        def row(r, c2):
            li = lid_v[pl.ds(r, 16)][0]

            def col(cix, c3):
                off = cix * 16
                plsc.addupdate(acc_v.at[li, pl.ds(off, 16)],
                               rows_v[r, pl.ds(off, 16)])
                return c3

            lax.fori_loop(0, H // 16, col, 0)
            return c2

        lax.fori_loop(0, FB, row, 0)
        return carry

    lax.fori_loop(0, nb, block, 0)

    @pl.when(wid < NW - 1)
    def _():
        pltpu.sync_copy(acc_v.at[pl.ds(0, OWN)], out_hbm.at[pl.ds(base, OWN)])

    @pl.when(wid == NW - 1)
    def _():
        pltpu.sync_copy(acc_v.at[pl.ds(0, OWN_LAST)],
                        out_hbm.at[pl.ds(base, OWN_LAST)])


@functools.partial(
    pl.kernel, mesh=_MESH,
    out_type=(jax.ShapeDtypeStruct((NW, G, H), jnp.float32),
              jax.ShapeDtypeStruct((NW, G, H), jnp.float32)),
    scratch_types=[pltpu.VMEM((CP + 16,), jnp.int32),
                   pltpu.VMEM((CP, H), jnp.float32),
                   pltpu.VMEM((G, H), jnp.float32),
                   pltpu.VMEM((G, H), jnp.float32)],
)
def _pool_k(h_hbm, bat_hbm, zg_hbm, ng_hbm, osum_hbm, omax_hbm,
            bat_v, rows_v, accs_v, accm_v):
    wid = lax.axis_index("s") * 2 + lax.axis_index("c")
    base = wid * PPW
    pltpu.sync_copy(zg_hbm, accs_v)
    pltpu.sync_copy(ng_hbm, accm_v)

    def accumulate(nbase, count):
        pltpu.sync_copy(h_hbm.at[pl.ds(nbase, count)], rows_v.at[pl.ds(0, count)])
        pltpu.sync_copy(bat_hbm.at[pl.ds(nbase, count)], bat_v.at[pl.ds(0, count)])

        def node_body(i, carry):
            g = bat_v[pl.ds(i, 16)][0]

            def col_body(cix, c2):
                off = cix * 16
                v = rows_v[i, pl.ds(off, 16)]
                accs_v[g, pl.ds(off, 16)] = accs_v[g, pl.ds(off, 16)] + v
                accm_v[g, pl.ds(off, 16)] = jnp.maximum(
                    accm_v[g, pl.ds(off, 16)], v)
                return c2

            lax.fori_loop(0, H // 16, col_body, 0)
            return carry

        lax.fori_loop(0, count, node_body, 0)

    def chunk_body(k, carry):
        accumulate(base + k * CP, CP)
        return carry

    lax.fori_loop(0, PPW // CP, chunk_body, 0)

    @pl.when(wid == NW - 1)
    def _():
        accumulate(NW * PPW, N - NW * PPW)

    pltpu.sync_copy(accs_v, osum_hbm.at[wid])
    pltpu.sync_copy(accm_v, omax_hbm.at[wid])


def kernel(x, edge_index, edge_attr, batch, W_emb, b_emb, W1, b1, W2, b2,
           W_ih, W_hh, b_ih, b_hh, bn_gamma, bn_beta):
    src = edge_index[0].astype(jnp.int32)
    dst = edge_index[1].astype(jnp.int32)
    bat = batch.astype(jnp.int32)

    W1a = W1[:, :H, :]
    W1b = W1[:, H:, :]
    W_ihT = jnp.swapaxes(W_ih, 1, 2)
    W_hhT = jnp.swapaxes(W_hh, 1, 2)

    zero_rows = jnp.zeros((CS, H), jnp.float32)
    zero_g = jnp.zeros((G, H), jnp.float32)
    neg_g = jnp.full((G, H), -jnp.inf, jnp.float32)

    elists, ecnts = _prep_lists(dst)
    h, p = _embed(x, W_emb, b_emb, W1a[0])
    for l in range(L):
        pj = _gather_k(p, src)
        m = _edge_mlp(pj, edge_attr, W1b[l], b1[l], W2[l], b2[l])
        aggr = _scatter_k(m, elists, ecnts, zero_rows)
        h, p = _gru(aggr, h, W_ihT[l], W_hhT[l], b_ih[l], b_hh[l],
                    bn_gamma[l], bn_beta[l], W1a[(l + 1) % L])

    psum, pmax = _pool_k(h, bat, zero_g, neg_g)
    return _combine(psum, pmax, bat.reshape(1, N))
